# baseline (speedup 1.0000x reference)
import jax
import jax.numpy as jnp
from jax.experimental import pallas as pl
from jax.experimental.pallas import tpu as pltpu

N = 35
HID = 120
H = 16
_NEG = -1e30
_S = 8


def _attn_head(h, countf, has_edge, a_src, a_dst, hd, C):
    f32 = jnp.float32
    hs = h[:, hd * C:(hd + 1) * C]
    asr = a_src[hd:hd + 1, :]
    adr = a_dst[hd:hd + 1, :]
    col = jax.lax.dot_general(
        hs, asr, (((1,), (1,)), ((), ())), preferred_element_type=f32)
    row = jax.lax.dot_general(
        adr, hs, (((1,), (1,)), ((), ())), preferred_element_type=f32)
    e = col + row
    e = jnp.where(e >= 0.0, e, 0.2 * e)
    e = jnp.where(has_edge, e, _NEG)
    m = jnp.max(e, axis=0, keepdims=True)
    ex = jnp.exp(e - m) * countf
    s = jnp.sum(ex, axis=0, keepdims=True)
    p = ex / (s + 1e-16)
    return jax.lax.dot_general(
        p, hs, (((0,), (0,)), ((), ())), preferred_element_type=f32)


def _elu(x):
    return jnp.where(x > 0.0, x, jnp.exp(jnp.minimum(x, 0.0)) - 1.0)


def _gat_kernel(adj_ref, W1_ref, as1_ref, ad1_ref, b1_ref, *rest):
    w2_refs = rest[:_S]
    (as2_ref, ad2_ref, b2_ref, W3_ref, as3_ref, ad3_ref, b3_ref,
     Wfc_ref, bfc_ref, out_ref, h1_s, h2_s) = rest[_S:]
    f32 = jnp.float32
    i = pl.program_id(0)

    adj = adj_ref[:]
    ii = jax.lax.broadcasted_iota(jnp.int32, (N, N), 0)
    jj = jax.lax.broadcasted_iota(jnp.int32, (N, N), 1)
    countf = (adj > 0.5).astype(f32) + (ii == jj).astype(f32)
    has_edge = countf > 0.0

    def half(first):
        if first:
            h1_s[:] = jnp.dot(adj, W1_ref[:], preferred_element_type=f32)
        h1 = h1_s[:]
        as1 = as1_ref[:]
        ad1 = ad1_ref[:]
        b1 = jnp.reshape(b1_ref[:], (1, H * HID))
        acc = None
        for p in range(_S):
            hd = (0 if first else _S) + p
            o = _attn_head(h1, countf, has_edge, as1, ad1, hd, HID)
            x1k = _elu(o + b1[:, hd * HID:(hd + 1) * HID])
            part = jnp.dot(x1k, w2_refs[p][:], preferred_element_type=f32)
            acc = part if acc is None else acc + part
        h2_s[:] = acc if first else h2_s[:] + acc

    @pl.when(i == 0)
    def _():
        half(True)

    @pl.when(i == 1)
    def _():
        half(False)

    @pl.when(i == 2)
    def _():
        h2 = h2_s[:]
        as2 = as2_ref[:]
        ad2 = ad2_ref[:]
        b2 = jnp.reshape(b2_ref[:], (1, H * HID))
        W3 = W3_ref[:]
        h3 = None
        for hd in range(H):
            o = _attn_head(h2, countf, has_edge, as2, ad2, hd, HID)
            x2k = _elu(o + b2[:, hd * HID:(hd + 1) * HID])
            part = jnp.dot(x2k, W3[hd * HID:(hd + 1) * HID, :],
                           preferred_element_type=f32)
            h3 = part if h3 is None else h3 + part
        o3 = _attn_head(h3, countf, has_edge, as3_ref[:], ad3_ref[:], 0, HID)
        x3 = o3 + jnp.reshape(b3_ref[:], (1, HID))
        out = (jnp.dot(x3, Wfc_ref[:], preferred_element_type=f32)
               + jnp.reshape(bfc_ref[:], (1, N)))
        out_ref[:] = jnp.maximum(out, 0.0)


def _full(shape):
    nd = len(shape)
    return pl.BlockSpec(shape, lambda i: (0,) * nd)


def kernel(adj_matrix, W1, as1, ad1, b1, W2, as2, ad2, b2,
           W3, as3, ad3, b3, Wfc, bfc):
    KC = H * HID
    w2_specs = [
        pl.BlockSpec((HID, KC),
                     lambda i, p=p: (jnp.minimum(i, 1) * _S + p, 0))
        for p in range(_S)
    ]
    in_specs = [
        _full((N, N)), _full((N, KC)), _full((H, HID)), _full((H, HID)),
        _full((KC,)),
        *w2_specs,
        _full((H, HID)), _full((H, HID)), _full((KC,)),
        _full((KC, HID)), _full((1, HID)), _full((1, HID)), _full((HID,)),
        _full((HID, N)), _full((N,)),
    ]
    return pl.pallas_call(
        _gat_kernel,
        out_shape=jax.ShapeDtypeStruct((N, N), jnp.float32),
        grid=(3,),
        in_specs=in_specs,
        out_specs=_full((N, N)),
        scratch_shapes=[
            pltpu.VMEM((N, KC), jnp.float32),
            pltpu.VMEM((N, KC), jnp.float32),
        ],
    )(adj_matrix, W1, as1, ad1, b1, *([W2] * _S), as2, ad2, b2,
      W3, as3, ad3, b3, Wfc, bfc)


# speedup vs baseline: 1.4408x; 1.4408x over previous
import jax
import jax.numpy as jnp
from jax.experimental import pallas as pl

N = 35
HID = 120
H = 16
_NEG = -1e30
_Q = 16
_QW = H * HID // _Q


def _gat_kernel(adj_ref, W1_ref, as1_ref, ad1_ref, b1_ref, *rest):
    w2_refs = rest[:_Q]
    (as2_ref, ad2_ref, b2_ref, W3_ref, as3_ref, ad3_ref, b3_ref,
     Wfc_ref, bfc_ref, out_ref) = rest[_Q:]
    f32 = jnp.float32
    adj = adj_ref[:]
    ii = jax.lax.broadcasted_iota(jnp.int32, (N, N), 0)
    jj = jax.lax.broadcasted_iota(jnp.int32, (N, N), 1)
    countf = (adj > 0.5).astype(f32) + (ii == jj).astype(f32)
    has_edge = countf > 0.0

    def heads_block(h, a_src, a_dst, head_ids, C):
        outs = []
        for k, hd in enumerate(head_ids):
            hs = h[:, k * C:(k + 1) * C]
            asr = a_src[hd:hd + 1, :]
            adr = a_dst[hd:hd + 1, :]
            col = jax.lax.dot_general(
                hs, asr, (((1,), (1,)), ((), ())), preferred_element_type=f32)
            row = jax.lax.dot_general(
                adr, hs, (((1,), (1,)), ((), ())), preferred_element_type=f32)
            e = col + row
            e = jnp.where(e >= 0.0, e, 0.2 * e)
            e = jnp.where(has_edge, e, _NEG)
            m = jnp.max(e, axis=0, keepdims=True)
            ex = jnp.exp(e - m) * countf
            s = jnp.sum(ex, axis=0, keepdims=True)
            p = ex / (s + 1e-16)
            outs.append(jax.lax.dot_general(
                p, hs, (((0,), (0,)), ((), ())), preferred_element_type=f32))
        return outs

    def elu(x):
        return jnp.where(x > 0.0, x, jnp.exp(jnp.minimum(x, 0.0)) - 1.0)

    h1 = jnp.dot(adj, W1_ref[:], preferred_element_type=f32)
    o1 = heads_block(h1, as1_ref[:], ad1_ref[:], list(range(H)), HID)
    x1 = elu(jnp.concatenate(o1, axis=1) + jnp.reshape(b1_ref[:], (1, H * HID)))

    h2 = None
    for q, wref in enumerate(w2_refs):
        part = jnp.dot(x1[:, q * _QW:(q + 1) * _QW], wref[:],
                       preferred_element_type=f32)
        h2 = part if h2 is None else h2 + part
    o2 = heads_block(h2, as2_ref[:], ad2_ref[:], list(range(H)), HID)
    x2 = elu(jnp.concatenate(o2, axis=1) + jnp.reshape(b2_ref[:], (1, H * HID)))

    h3 = jnp.dot(x2, W3_ref[:], preferred_element_type=f32)
    o3 = heads_block(h3, as3_ref[:], ad3_ref[:], [0], HID)[0]
    x3 = o3 + jnp.reshape(b3_ref[:], (1, HID))
    out = (jnp.dot(x3, Wfc_ref[:], preferred_element_type=f32)
           + jnp.reshape(bfc_ref[:], (1, N)))
    out_ref[:] = jnp.maximum(out, 0.0)


def _full(shape):
    nd = len(shape)
    return pl.BlockSpec(shape, lambda i: (0,) * nd)


def kernel(adj_matrix, W1, as1, ad1, b1, W2, as2, ad2, b2,
           W3, as3, ad3, b3, Wfc, bfc):
    KC = H * HID
    w2_specs = [pl.BlockSpec((_QW, KC), lambda i, q=q: (q, 0))
                for q in range(_Q)]
    in_specs = [
        _full((N, N)), _full((N, KC)), _full((H, HID)), _full((H, HID)),
        _full((KC,)),
        *w2_specs,
        _full((H, HID)), _full((H, HID)), _full((KC,)),
        _full((KC, HID)), _full((1, HID)), _full((1, HID)), _full((HID,)),
        _full((HID, N)), _full((N,)),
    ]
    return pl.pallas_call(
        _gat_kernel,
        out_shape=jax.ShapeDtypeStruct((N, N), jnp.float32),
        grid=(1,),
        in_specs=in_specs,
        out_specs=_full((N, N)),
    )(adj_matrix, W1, as1, ad1, b1, *([W2] * _Q), as2, ad2, b2,
      W3, as3, ad3, b3, Wfc, bfc)


# speedup vs baseline: 1.4951x; 1.0377x over previous
import jax
import jax.numpy as jnp
from jax.experimental import pallas as pl
from jax.experimental.pallas import tpu as pltpu

N = 35
HID = 120
H = 16
_NEG = -1e30
_NS = 15
_SW = 128


def _gat_kernel(adj_ref, W1_ref, as1_ref, ad1_ref, b1_ref, W2_hbm,
                as2_ref, ad2_ref, b2_ref, W3_ref, as3_ref, ad3_ref, b3_ref,
                Wfc_ref, bfc_ref, out_ref, w2_vmem, sems):
    f32 = jnp.float32

    def slab_copy(q):
        return pltpu.make_async_copy(
            W2_hbm.at[pl.ds(q * _SW, _SW), :],
            w2_vmem.at[pl.ds(q * _SW, _SW), :],
            sems.at[q])

    for q in range(_NS):
        slab_copy(q).start()

    adj = adj_ref[:]
    ii = jax.lax.broadcasted_iota(jnp.int32, (N, N), 0)
    jj = jax.lax.broadcasted_iota(jnp.int32, (N, N), 1)
    countf = (adj > 0.5).astype(f32) + (ii == jj).astype(f32)
    has_edge = countf > 0.0

    def heads_block(h, a_src, a_dst, head_ids, C):
        outs = []
        for k, hd in enumerate(head_ids):
            hs = h[:, k * C:(k + 1) * C]
            asr = a_src[hd:hd + 1, :]
            adr = a_dst[hd:hd + 1, :]
            col = jax.lax.dot_general(
                hs, asr, (((1,), (1,)), ((), ())), preferred_element_type=f32)
            row = jax.lax.dot_general(
                adr, hs, (((1,), (1,)), ((), ())), preferred_element_type=f32)
            e = col + row
            e = jnp.where(e >= 0.0, e, 0.2 * e)
            e = jnp.where(has_edge, e, _NEG)
            m = jnp.max(e, axis=0, keepdims=True)
            ex = jnp.exp(e - m) * countf
            s = jnp.sum(ex, axis=0, keepdims=True)
            p = ex / (s + 1e-16)
            outs.append(jax.lax.dot_general(
                p, hs, (((0,), (0,)), ((), ())), preferred_element_type=f32))
        return outs

    def elu(x):
        return jnp.where(x > 0.0, x, jnp.exp(jnp.minimum(x, 0.0)) - 1.0)

    h1 = jnp.dot(adj, W1_ref[:], preferred_element_type=f32)
    o1 = heads_block(h1, as1_ref[:], ad1_ref[:], list(range(H)), HID)
    x1 = elu(jnp.concatenate(o1, axis=1) + jnp.reshape(b1_ref[:], (1, H * HID)))

    h2 = None
    for q in range(_NS):
        slab_copy(q).wait()
        part = jnp.dot(x1[:, q * _SW:(q + 1) * _SW],
                       w2_vmem[q * _SW:(q + 1) * _SW, :],
                       preferred_element_type=f32)
        h2 = part if h2 is None else h2 + part
    o2 = heads_block(h2, as2_ref[:], ad2_ref[:], list(range(H)), HID)
    x2 = elu(jnp.concatenate(o2, axis=1) + jnp.reshape(b2_ref[:], (1, H * HID)))

    h3 = jnp.dot(x2, W3_ref[:], preferred_element_type=f32)
    o3 = heads_block(h3, as3_ref[:], ad3_ref[:], [0], HID)[0]
    x3 = o3 + jnp.reshape(b3_ref[:], (1, HID))
    out = (jnp.dot(x3, Wfc_ref[:], preferred_element_type=f32)
           + jnp.reshape(bfc_ref[:], (1, N)))
    out_ref[:] = jnp.maximum(out, 0.0)


def _full(shape):
    nd = len(shape)
    return pl.BlockSpec(shape, lambda i: (0,) * nd)


def kernel(adj_matrix, W1, as1, ad1, b1, W2, as2, ad2, b2,
           W3, as3, ad3, b3, Wfc, bfc):
    KC = H * HID
    in_specs = [
        _full((N, N)), _full((N, KC)), _full((H, HID)), _full((H, HID)),
        _full((KC,)),
        pl.BlockSpec(memory_space=pltpu.MemorySpace.HBM),
        _full((H, HID)), _full((H, HID)), _full((KC,)),
        _full((KC, HID)), _full((1, HID)), _full((1, HID)), _full((HID,)),
        _full((HID, N)), _full((N,)),
    ]
    return pl.pallas_call(
        _gat_kernel,
        out_shape=jax.ShapeDtypeStruct((N, N), jnp.float32),
        grid=(1,),
        in_specs=in_specs,
        out_specs=_full((N, N)),
        scratch_shapes=[
            pltpu.VMEM((KC, KC), jnp.float32),
            pltpu.SemaphoreType.DMA((_NS,)),
        ],
    )(adj_matrix, W1, as1, ad1, b1, W2, as2, ad2, b2,
      W3, as3, ad3, b3, Wfc, bfc)
